# baseline (speedup 1.0000x reference)
import jax
import jax.numpy as jnp
from jax import lax
from jax.experimental import pallas as pl
from jax.experimental.pallas import tpu as pltpu
from jax.experimental.pallas import tpu_sc as plsc

D_MODEL = 128
W = 128
WINDOWS = 200
GROUP = 3
BLOCK = 48
PHASES = 4
TAILW = WINDOWS - PHASES * BLOCK
NW = 32


def kernel(x, p2e):
    shp = x.shape
    idx = jnp.reshape(x, (NW, WINDOWS, W)).astype(jnp.int32)
    mesh = plsc.VectorSubcoreMesh(core_axis_name="core", subcore_axis_name="subcore")

    @pl.kernel(
        out_type=jax.ShapeDtypeStruct((NW, WINDOWS, W, D_MODEL), p2e.dtype),
        mesh=mesh,
        scratch_types=[
            pltpu.VMEM_SHARED(p2e.shape, p2e.dtype),
            pltpu.VMEM((BLOCK, W), jnp.int32),
            pltpu.VMEM((BLOCK, W), jnp.int32),
            pltpu.VMEM((W, D_MODEL), p2e.dtype),
            pltpu.VMEM((W, D_MODEL), p2e.dtype),
            pltpu.VMEM((W, D_MODEL), p2e.dtype),
            pltpu.SemaphoreType.DMA,
            pltpu.SemaphoreType.DMA,
            pltpu.SemaphoreType.DMA,
            pltpu.SemaphoreType.DMA,
            pltpu.SemaphoreType.DMA,
            pltpu.SemaphoreType.DMA,
            pltpu.SemaphoreType.DMA,
        ],
    )
    def gather_kernel(
        table_hbm, idx_hbm, out_hbm, table_spmem, ibuf0, ibuf1,
        buf0, buf1, buf2, g0, g1, g2, w0, w1, w2, isem,
    ):
        cid = lax.axis_index("core")
        sid = lax.axis_index("subcore")
        wid = sid * 2 + cid

        @pl.when(sid == 0)
        def _():
            pltpu.sync_copy(table_hbm, table_spmem)

        pltpu.sync_copy(idx_hbm.at[wid, pl.ds(0, BLOCK)], ibuf0)
        plsc.subcore_barrier()

        ibufs = (ibuf0, ibuf1)
        bufs = (buf0, buf1, buf2)
        gsems = (g0, g1, g2)
        wsems = (w0, w1, w2)
        srcs = (table_spmem, table_spmem, table_hbm)

        def start_gather(ib, r, b):
            return pltpu.async_copy(srcs[b].at[ib.at[r]], bufs[b], gsems[b])

        def wait_write(b):
            pltpu.make_async_copy(bufs[b], out_hbm.at[wid, 0], wsems[b]).wait()

        def start_write(s, b):
            pltpu.async_copy(bufs[b], out_hbm.at[wid, s], wsems[b])

        def do_group(ib, base_w, g, first):
            handles = []
            for b in range(GROUP):
                if not first:
                    wait_write(b)
                handles.append(start_gather(ib, g * GROUP + b, b))
            for b in range(GROUP):
                handles[b].wait()
                start_write(base_w + g * GROUP + b, b)

        for p in range(PHASES):
            ib = ibufs[p % 2]
            nxt = ibufs[(p + 1) % 2]
            if p < PHASES - 1:
                pltpu.async_copy(
                    idx_hbm.at[wid, pl.ds((p + 1) * BLOCK, BLOCK)], nxt, isem
                )
            else:
                pltpu.async_copy(
                    idx_hbm.at[wid, pl.ds(PHASES * BLOCK, TAILW)],
                    nxt.at[pl.ds(0, TAILW)],
                    isem,
                )
            if p > 0:
                pltpu.make_async_copy(
                    idx_hbm.at[wid, pl.ds(0, BLOCK)], ib, isem
                ).wait()

            if p == 0:
                do_group(ib, 0, 0, first=True)

                @pl.loop(1, BLOCK // GROUP)
                def _(g):
                    do_group(ib, 0, g, first=False)
            else:
                base_w = p * BLOCK

                @pl.loop(0, BLOCK // GROUP)
                def _(g, base_w=base_w, ib=ib):
                    do_group(ib, base_w, g, first=False)

        tail_ib = ibufs[PHASES % 2]
        pltpu.make_async_copy(
            idx_hbm.at[wid, pl.ds(0, TAILW)], tail_ib.at[pl.ds(0, TAILW)], isem
        ).wait()
        base_w = PHASES * BLOCK
        for g in range(2):
            do_group(tail_ib, base_w, g, first=False)
        tail_handles = {}
        for b, r in ((0, 6), (2, 7)):
            wait_write(b)
            tail_handles[b] = start_gather(tail_ib, r, b)
        for b, r in ((0, 6), (2, 7)):
            tail_handles[b].wait()
            start_write(base_w + r, b)

        for b in range(GROUP):
            wait_write(b)

    out = gather_kernel(p2e, idx)
    return jnp.reshape(out, shp + (D_MODEL,))


# speedup vs baseline: 1.2927x; 1.2927x over previous
import jax
import jax.numpy as jnp
from jax import lax
from jax.experimental import pallas as pl
from jax.experimental.pallas import tpu as pltpu
from jax.experimental.pallas import tpu_sc as plsc

D_MODEL = 128
W = 128
WINDOWS = 200
GROUP = 3
BLOCK = 48
PHASES = 4
TAILW = WINDOWS - PHASES * BLOCK
NW = 32


def kernel(x, p2e):
    shp = x.shape
    idx = jnp.reshape(x, (NW, WINDOWS, W)).astype(jnp.int32)
    mesh = plsc.VectorSubcoreMesh(core_axis_name="core", subcore_axis_name="subcore")

    @pl.kernel(
        out_type=jax.ShapeDtypeStruct((NW, WINDOWS, W, D_MODEL), p2e.dtype),
        mesh=mesh,
        scratch_types=[
            pltpu.VMEM_SHARED(p2e.shape, p2e.dtype),
            pltpu.VMEM((BLOCK, W), jnp.int32),
            pltpu.VMEM((BLOCK, W), jnp.int32),
            pltpu.VMEM((W, D_MODEL), p2e.dtype),
            pltpu.VMEM((W, D_MODEL), p2e.dtype),
            pltpu.VMEM((W, D_MODEL), p2e.dtype),
            pltpu.SemaphoreType.DMA,
            pltpu.SemaphoreType.DMA,
            pltpu.SemaphoreType.DMA,
            pltpu.SemaphoreType.DMA,
            pltpu.SemaphoreType.DMA,
            pltpu.SemaphoreType.DMA,
            pltpu.SemaphoreType.DMA,
        ],
    )
    def gather_kernel(
        table_hbm, idx_hbm, out_hbm, table_spmem, ibuf0, ibuf1,
        buf0, buf1, buf2, g0, g1, g2, w0, w1, w2, isem,
    ):
        cid = lax.axis_index("core")
        sid = lax.axis_index("subcore")
        wid = sid * 2 + cid

        @pl.when(sid == 0)
        def _():
            pltpu.sync_copy(table_hbm, table_spmem)

        pltpu.sync_copy(idx_hbm.at[wid, pl.ds(0, BLOCK)], ibuf0)
        plsc.subcore_barrier()

        ibufs = (ibuf0, ibuf1)
        bufs = (buf0, buf1, buf2)
        gsems = (g0, g1, g2)
        wsems = (w0, w1, w2)
        srcs = (table_spmem, table_spmem, table_spmem)

        def start_gather(ib, r, b):
            return pltpu.async_copy(srcs[b].at[ib.at[r]], bufs[b], gsems[b])

        def wait_write(b):
            pltpu.make_async_copy(bufs[b], out_hbm.at[wid, 0], wsems[b]).wait()

        def start_write(s, b):
            pltpu.async_copy(bufs[b], out_hbm.at[wid, s], wsems[b])

        def do_group(ib, base_w, g, first):
            handles = []
            for b in range(GROUP):
                if not first:
                    wait_write(b)
                handles.append(start_gather(ib, g * GROUP + b, b))
            for b in range(GROUP):
                handles[b].wait()
                start_write(base_w + g * GROUP + b, b)

        for p in range(PHASES):
            ib = ibufs[p % 2]
            nxt = ibufs[(p + 1) % 2]
            if p < PHASES - 1:
                pltpu.async_copy(
                    idx_hbm.at[wid, pl.ds((p + 1) * BLOCK, BLOCK)], nxt, isem
                )
            else:
                pltpu.async_copy(
                    idx_hbm.at[wid, pl.ds(PHASES * BLOCK, TAILW)],
                    nxt.at[pl.ds(0, TAILW)],
                    isem,
                )
            if p > 0:
                pltpu.make_async_copy(
                    idx_hbm.at[wid, pl.ds(0, BLOCK)], ib, isem
                ).wait()

            if p == 0:
                do_group(ib, 0, 0, first=True)

                @pl.loop(1, BLOCK // GROUP)
                def _(g):
                    do_group(ib, 0, g, first=False)
            else:
                base_w = p * BLOCK

                @pl.loop(0, BLOCK // GROUP)
                def _(g, base_w=base_w, ib=ib):
                    do_group(ib, base_w, g, first=False)

        tail_ib = ibufs[PHASES % 2]
        pltpu.make_async_copy(
            idx_hbm.at[wid, pl.ds(0, TAILW)], tail_ib.at[pl.ds(0, TAILW)], isem
        ).wait()
        base_w = PHASES * BLOCK
        for g in range(2):
            do_group(tail_ib, base_w, g, first=False)
        tail_handles = {}
        for b, r in ((0, 6), (2, 7)):
            wait_write(b)
            tail_handles[b] = start_gather(tail_ib, r, b)
        for b, r in ((0, 6), (2, 7)):
            tail_handles[b].wait()
            start_write(base_w + r, b)

        for b in range(GROUP):
            wait_write(b)

    out = gather_kernel(p2e, idx)
    return jnp.reshape(out, shp + (D_MODEL,))
